# Initial kernel scaffold; baseline (speedup 1.0000x reference)
import jax
import jax.numpy as jnp
from jax.experimental import pallas as pl


def kernel(x, edge_index, W_self, W_neigh, b):
    raise NotImplementedError("write your pallas kernel here")


# speedup vs baseline: 6.3376x; 6.3376x over previous
import functools

import jax
import jax.numpy as jnp
from jax import lax
from jax.experimental import pallas as pl
from jax.experimental.pallas import tpu as pltpu
from jax.experimental.pallas import tpu_sc as plsc

N_NODES = 10000
D = 128
DH = D // 2
E = 320000
NC = 2
NS = 16
EPS = E // NS
K = 128
CHUNKS = -(-EPS // K)
EPS_PAD = CHUNKS * K
ACC_N = 10240
RPS = ACC_N // NS
CW = 16


def _zero_fill(ref, nrows, ncols):
    z = jnp.zeros((16,), jnp.float32)

    def body(r, carry):
        for c in range(ncols // 16):
            ref[r, pl.ds(c * 16, 16)] = z
        return carry

    lax.fori_loop(0, nrows, body, 0)


def _sc_segment_sum(x0, x1, src_idx, dst_idx):
    mesh = plsc.VectorSubcoreMesh(core_axis_name="c", subcore_axis_name="s")

    @functools.partial(
        pl.kernel,
        mesh=mesh,
        compiler_params=pltpu.CompilerParams(use_tc_tiling_on_sc=False),
        out_type=(
            jax.ShapeDtypeStruct((NC, ACC_N, DH), jnp.float32),
            jax.ShapeDtypeStruct((ACC_N, CW), jnp.float32),
        ),
        scratch_types=[
            pltpu.VMEM((CHUNKS, K), jnp.int32),
            pltpu.VMEM((CHUNKS, K), jnp.int32),
            pltpu.VMEM((K, DH), jnp.float32),
            pltpu.VMEM((K, CW), jnp.float32),
            pltpu.VMEM_SHARED((ACC_N, DH), jnp.float32),
            pltpu.VMEM_SHARED((ACC_N, CW), jnp.float32),
            pltpu.SemaphoreType.DMA,
        ],
    )
    def k(x0_hbm, x1_hbm, src_hbm, dst_hbm, out_sum, out_cnt,
          src_v, dst_v, rows_v, ones_v, acc_sh, cnt_sh, sem):
        cid = lax.axis_index("c")
        sid = lax.axis_index("s")

        pltpu.sync_copy(src_hbm.at[sid], src_v)
        pltpu.sync_copy(dst_hbm.at[sid], dst_v)

        _zero_fill(rows_v, K, DH)
        _zero_fill(ones_v, K, CW)
        base = sid * RPS
        for i in range(RPS // K):
            pltpu.sync_copy(rows_v, acc_sh.at[pl.ds(base + i * K, K)])
            pltpu.sync_copy(ones_v, cnt_sh.at[pl.ds(base + i * K, K)])

        one = jnp.ones((16,), jnp.float32)

        def fill_ones(r, carry):
            ones_v[r, pl.ds(0, 16)] = one
            return carry

        lax.fori_loop(0, K, fill_ones, 0)
        plsc.subcore_barrier()

        def step(j, carry):
            @pl.when(cid == 0)
            def _():
                pltpu.async_copy(x0_hbm.at[src_v.at[j]], rows_v, sem).wait()

            @pl.when(cid == 1)
            def _():
                pltpu.async_copy(x1_hbm.at[src_v.at[j]], rows_v, sem).wait()

            pltpu.sync_copy(rows_v, acc_sh.at[dst_v.at[j]], add=True)

            @pl.when(cid == 0)
            def _():
                pltpu.sync_copy(ones_v, cnt_sh.at[dst_v.at[j]], add=True)

            return carry

        lax.fori_loop(0, CHUNKS, step, 0)
        plsc.subcore_barrier()

        pltpu.sync_copy(acc_sh.at[pl.ds(base, RPS)],
                        out_sum.at[cid, pl.ds(base, RPS)])

        @pl.when(cid == 0)
        def _():
            pltpu.sync_copy(cnt_sh.at[pl.ds(base, RPS)],
                            out_cnt.at[pl.ds(base, RPS)])

    return k(x0, x1, src_idx, dst_idx)


def _tc_combine(x, sums, cnts, W_self, W_neigh, b2d):
    R = 400
    grid = N_NODES // R

    def body(x_ref, s0_ref, s1_ref, c_ref, ws_ref, wn_ref, b_ref, o_ref):
        cnt = jnp.maximum(c_ref[:, 0:1], 1.0)
        mean0 = s0_ref[0] / cnt
        mean1 = s1_ref[0] / cnt
        hi = jax.lax.Precision.HIGHEST
        o_ref[...] = (
            jax.lax.dot(x_ref[...], ws_ref[...], precision=hi,
                        preferred_element_type=jnp.float32)
            + jax.lax.dot(mean0, wn_ref[0:DH, :], precision=hi,
                          preferred_element_type=jnp.float32)
            + jax.lax.dot(mean1, wn_ref[DH:D, :], precision=hi,
                          preferred_element_type=jnp.float32)
            + b_ref[...])

    return pl.pallas_call(
        body,
        grid=(grid,),
        in_specs=[
            pl.BlockSpec((R, D), lambda i: (i, 0)),
            pl.BlockSpec((1, R, DH), lambda i: (0, i, 0)),
            pl.BlockSpec((1, R, DH), lambda i: (1, i, 0)),
            pl.BlockSpec((R, CW), lambda i: (i, 0)),
            pl.BlockSpec((D, D), lambda i: (0, 0)),
            pl.BlockSpec((D, D), lambda i: (0, 0)),
            pl.BlockSpec((1, D), lambda i: (0, 0)),
        ],
        out_specs=pl.BlockSpec((R, D), lambda i: (i, 0)),
        out_shape=jax.ShapeDtypeStruct((N_NODES, D), jnp.float32),
    )(x, sums, sums, cnts, W_self, W_neigh, b2d)


def kernel(x, edge_index, W_self, W_neigh, b):
    src = edge_index[0].astype(jnp.int32).reshape(NS, EPS)
    dst = edge_index[1].astype(jnp.int32).reshape(NS, EPS)
    pad = EPS_PAD - EPS
    src = jnp.pad(src, ((0, 0), (0, pad))).reshape(NS, CHUNKS, K)
    dst = jnp.pad(dst, ((0, 0), (0, pad)),
                  constant_values=N_NODES).reshape(NS, CHUNKS, K)
    x0 = x[:, :DH]
    x1 = x[:, DH:]
    sums, cnts = _sc_segment_sum(x0, x1, src, dst)
    return _tc_combine(x, sums, cnts, W_self, W_neigh, b.reshape(1, D))
